# baseline (speedup 1.0000x reference)
import functools

import jax
import jax.numpy as jnp
from jax import lax
from jax.experimental import pallas as pl
from jax.experimental.pallas import tpu as pltpu
from jax.experimental.pallas import tpu_sc as plsc

B, K, N = 8, 4, 10
BATCHES = B * K
C, H, W = 384, 32, 32
NBOX = B * K * N
NW = 32
BOX_PER_W = NBOX // NW
IDX_PER_W = BOX_PER_W * C
CHW = C * H * W
ROWS_PER_CH = H * W // 16
CHUNK = 128
NCHUNK = IDX_PER_W // CHUNK
OUT_ROWS_W = IDX_PER_W // 16


def _body(table_hbm, bboxes_hbm, out_hbm, bb_v, rb_v, off_v, idx_v,
          rows_v, out_v, sem):
    wid = lax.axis_index("s") * 2 + lax.axis_index("c")

    pltpu.sync_copy(bboxes_hbm, bb_v)

    lane = lax.broadcasted_iota(jnp.int32, (16,), 0)
    box = jnp.minimum(wid * BOX_PER_W + lane, NBOX - 1)

    def field(f):
        p = box * 4 + f
        return plsc.load_gather(bb_v, [p >> 4, p & 15])

    x0, y0, bw, bh = field(0), field(1), field(2), field(3)
    cx = (x0 + (bw >> 1)) >> 4
    cy = (y0 + (bh >> 1)) >> 4
    rb = wid * (CHW // 16) + cy * (W // 16) + (cx >> 4)
    off = cx & 15
    rb_v[pl.ds(0, 16)] = rb
    rb_v[pl.ds(16, 16)] = rb
    off_v[pl.ds(0, 16)] = off
    off_v[pl.ds(16, 16)] = off

    for b in range(BOX_PER_W):
        rb_b = plsc.load_gather(rb_v, [jnp.full((16,), 16 + b, jnp.int32)])
        for j in range(C // 16):
            idx_v[pl.ds(b * C + j * 16, 16)] = (
                rb_b + lane * ROWS_PER_CH + j * (16 * ROWS_PER_CH))

    for j in range(NCHUNK):
        pltpu.async_copy(
            table_hbm.at[idx_v.at[pl.ds(j * CHUNK, CHUNK)]],
            rows_v.at[pl.ds(j * CHUNK, CHUNK)], sem)
    pltpu.make_async_copy(
        table_hbm.at[pl.ds(0, IDX_PER_W)], rows_v, sem).wait()

    for b in range(BOX_PER_W):
        ob = plsc.load_gather(off_v, [jnp.full((16,), 16 + b, jnp.int32)])
        for j in range(C // 16):
            out_v[b * (C // 16) + j, :] = plsc.load_gather(
                rows_v, [b * C + j * 16 + lane, ob])

    pltpu.sync_copy(out_v, out_hbm.at[pl.ds(wid * OUT_ROWS_W, OUT_ROWS_W)])


@jax.jit
def _center_pool(input, bboxes):
    mesh = plsc.VectorSubcoreMesh(core_axis_name="c", subcore_axis_name="s")
    run = functools.partial(
        pl.kernel,
        mesh=mesh,
        out_type=jax.ShapeDtypeStruct((NBOX * C // 16, 16), jnp.float32),
        scratch_types=[
            pltpu.VMEM((NBOX * 4 // 16, 16), jnp.int32),
            pltpu.VMEM((32,), jnp.int32),
            pltpu.VMEM((32,), jnp.int32),
            pltpu.VMEM((IDX_PER_W,), jnp.int32),
            pltpu.VMEM((IDX_PER_W, 16), jnp.float32),
            pltpu.VMEM((OUT_ROWS_W, 16), jnp.float32),
            pltpu.SemaphoreType.DMA,
        ],
        compiler_params=pltpu.CompilerParams(
            needs_layout_passes=False, use_tc_tiling_on_sc=False
        ),
    )(_body)
    out2d = run(input.reshape(BATCHES * CHW // 16, 16),
                bboxes.reshape(NBOX * 4 // 16, 16))
    return out2d.reshape(B, K * N, C)


def kernel(input, bboxes):
    return _center_pool(input, bboxes)


# speedup vs baseline: 1.0234x; 1.0234x over previous
import functools

import jax
import jax.numpy as jnp
from jax import lax
from jax.experimental import pallas as pl
from jax.experimental.pallas import tpu as pltpu
from jax.experimental.pallas import tpu_sc as plsc

B, K, N = 8, 4, 10
BATCHES = B * K
C, H, W = 384, 32, 32
NBOX = B * K * N
NW = 32
BOX_PER_W = NBOX // NW
IDX_PER_W = BOX_PER_W * C
CHW = C * H * W
HW = H * W
CHUNK = 128
NCHUNK = IDX_PER_W // CHUNK


def _body(flat_hbm, bboxes_hbm, out_hbm, bb_v, rb_v, idx_v, vals_v, sem):
    wid = lax.axis_index("s") * 2 + lax.axis_index("c")

    pltpu.sync_copy(bboxes_hbm, bb_v)

    lane = lax.broadcasted_iota(jnp.int32, (16,), 0)
    box = jnp.minimum(wid * BOX_PER_W + lane, NBOX - 1)

    def field(f):
        p = box * 4 + f
        return plsc.load_gather(bb_v, [p >> 4, p & 15])

    x0, y0, bw, bh = field(0), field(1), field(2), field(3)
    cx = (x0 + (bw >> 1)) >> 4
    cy = (y0 + (bh >> 1)) >> 4
    rb = wid * CHW + cy * W + cx
    rb_v[pl.ds(0, 16)] = rb
    rb_v[pl.ds(16, 16)] = rb

    for b in range(BOX_PER_W):
        rb_b = plsc.load_gather(rb_v, [jnp.full((16,), 16 + b, jnp.int32)])
        for j in range(C // 16):
            idx_v[pl.ds(b * C + j * 16, 16)] = (
                rb_b + lane * HW + j * (16 * HW))

    for j in range(NCHUNK):
        pltpu.async_copy(
            flat_hbm.at[idx_v.at[pl.ds(j * CHUNK, CHUNK)]],
            vals_v.at[pl.ds(j * CHUNK, CHUNK)], sem)
    pltpu.make_async_copy(
        flat_hbm.at[pl.ds(0, IDX_PER_W)], vals_v, sem).wait()

    pltpu.sync_copy(vals_v, out_hbm.at[pl.ds(wid * IDX_PER_W, IDX_PER_W)])


@jax.jit
def _center_pool(input, bboxes):
    mesh = plsc.VectorSubcoreMesh(core_axis_name="c", subcore_axis_name="s")
    run = functools.partial(
        pl.kernel,
        mesh=mesh,
        out_type=jax.ShapeDtypeStruct((NBOX * C,), jnp.float32),
        scratch_types=[
            pltpu.VMEM((NBOX * 4 // 16, 16), jnp.int32),
            pltpu.VMEM((32,), jnp.int32),
            pltpu.VMEM((IDX_PER_W,), jnp.int32),
            pltpu.VMEM((IDX_PER_W,), jnp.float32),
            pltpu.SemaphoreType.DMA,
        ],
        compiler_params=pltpu.CompilerParams(
            needs_layout_passes=False, use_tc_tiling_on_sc=False
        ),
    )(_body)
    out = run(input.reshape(BATCHES * CHW),
              bboxes.reshape(NBOX * 4 // 16, 16))
    return out.reshape(B, K * N, C)


def kernel(input, bboxes):
    return _center_pool(input, bboxes)
